# baseline (speedup 1.0000x reference)
import functools

import jax
import jax.numpy as jnp
from jax.experimental import pallas as pl
from jax.experimental.pallas import tpu as pltpu

_NUM_ETA = 19


def _body(mask_ref, eta_ref, z_ref, o_ref):
    zb = z_ref[0]
    mask = mask_ref[0]
    shp = mask.shape
    A = jnp.zeros(shp, jnp.float32)
    B = jnp.zeros(shp, jnp.float32)
    C = jnp.zeros(shp, jnp.float32)
    D = jnp.zeros(shp, jnp.float32)
    for t in range(_NUM_ETA):
        sel = mask == t
        e0 = eta_ref[t, 0]
        e1 = eta_ref[t, 1]
        e2 = eta_ref[t, 2]
        e3 = eta_ref[t, 3]
        A = jnp.where(sel, -e0, A)
        B = jnp.where(sel, -e1, B)
        C = jnp.where(sel, e3, C)
        D = jnp.where(sel, e2 * e3, D)
    o_ref[0] = A + B * jnp.tanh(zb * C - D)


@functools.partial(jax.jit, static_argnames=("interpret",))
def kernel(z, Mask, eta_table, interpret=False):
    M, N, S, F = z.shape
    W = S * F
    NB = 128
    zr = z.reshape(M, N, W)
    mask_lane = jnp.tile(Mask.astype(jnp.int32), (1, S)).reshape(M, 1, W)
    out = pl.pallas_call(
        _body,
        grid=(M, N // NB),
        in_specs=[
            pl.BlockSpec((1, 1, W), lambda m, n: (m, 0, 0)),
            pl.BlockSpec(memory_space=pltpu.SMEM),
            pl.BlockSpec((1, NB, W), lambda m, n: (m, n, 0)),
        ],
        out_specs=pl.BlockSpec((1, NB, W), lambda m, n: (m, n, 0)),
        out_shape=jax.ShapeDtypeStruct((M, N, W), jnp.float32),
        interpret=interpret,
    )(mask_lane, eta_table, zr)
    return out.reshape(M, N, S, F)
